# baseline (speedup 1.0000x reference)
import jax
import jax.numpy as jnp
from jax import lax
from jax.experimental import pallas as pl
from jax.experimental.pallas import tpu as pltpu
from jax.experimental.pallas import tpu_sc as plsc
from jax._src.pallas import mpmd as _pl_mpmd

B = 16384
C = 1000
NC = 2
NS = 16
NW = NC * NS
RPW = B // NW
ZBLK = 1024
TBL = 16384


def _zero_body(t_ref, o_ref):
    o_ref[...] = jnp.zeros_like(o_ref)


_zero_fill = pl.pallas_call(
    _zero_body,
    out_shape=jax.ShapeDtypeStruct((C, B), jnp.float32),
    grid=(B // ZBLK,),
    in_specs=[pl.BlockSpec(memory_space=pl.ANY)],
    out_specs=pl.BlockSpec((C, ZBLK), lambda i: (0, i)),
)


def _ones_body(tgt_hbm, zeros_hbm, out_hbm, idx_v, table_v, drain_v, sem):
    del zeros_hbm
    cid = lax.axis_index("c")
    sid = lax.axis_index("s")
    wid = sid * NC + cid
    base = pl.multiple_of(wid * RPW, 8)

    pltpu.sync_copy(tgt_hbm.at[pl.ds(base, RPW)], idx_v)

    @pl.loop(0, RPW // 16)
    def _grp(g):
        c16 = idx_v[pl.ds(pl.multiple_of(g * 16, 16), 16)]
        for k in range(16):
            ck = c16[k]
            q = jnp.where(c16 == ck, 1.0, 0.0)
            off = pl.multiple_of((g * 16 + k) * 16, 16)
            table_v[pl.ds(off, 16)] = q
            src = table_v.at[pl.ds(pl.multiple_of(off + (k & 8), 8), 8)]
            colw = pl.multiple_of(base + g * 16 + (k & ~7), 8)
            pltpu.async_copy(src, out_hbm.at[ck, pl.ds(colw, 8)], sem)

    pltpu.make_async_copy(tgt_hbm.at[pl.ds(0, 4096)], drain_v, sem).wait()


_sc_mesh = plsc.VectorSubcoreMesh(core_axis_name="c", subcore_axis_name="s")

_sc_ones = _pl_mpmd._mpmd_map(
    [(_sc_mesh, _ones_body)],
    jax.ShapeDtypeStruct((C, B), jnp.float32),
    input_output_aliases={1: 0},
    scratch_types=[
        pltpu.VMEM((RPW,), jnp.int32),
        pltpu.VMEM((TBL,), jnp.float32),
        pltpu.VMEM((4096,), jnp.int32),
        pltpu.SemaphoreType.DMA,
    ],
    compiler_params=pltpu.CompilerParams(needs_layout_passes=False),
    interpret=False,
    debug=False,
    cost_estimate=None,
    name="sc_one_hot_scatter",
    metadata=None,
)


def kernel(target):
    tgt = target.astype(jnp.int32)
    return _sc_ones(tgt, _zero_fill(tgt)).T
